# baseline (speedup 1.0000x reference)
import jax
import jax.numpy as jnp
from jax import lax
from jax.experimental import pallas as pl
from jax.experimental.pallas import tpu as pltpu
from jax.experimental.pallas import tpu_sc as plsc

B, T, D = 4, 4096, 2048
NC, NS = 2, 16
NW = NC * NS
TR = T // NW
CT = 16
NCH = TR // CT
CHUNK = CT * D


def _sc_body(x_hbm, t_hbm, o_hbm, tbuf, xbuf):
    wid = lax.axis_index("s") * NC + lax.axis_index("c")
    base = wid * (TR * D)

    def chunk_loop(c, carry):
        off = base + c * CHUNK
        pltpu.sync_copy(t_hbm.at[pl.ds(off, CHUNK)], tbuf)
        for b in range(B):
            pltpu.sync_copy(x_hbm.at[b, pl.ds(off, CHUNK)], xbuf)

            def add8(i, carry2):
                for u in range(8):
                    s = (i * 8 + u) * 16
                    plsc.addupdate(xbuf.at[pl.ds(s, 16)], tbuf[pl.ds(s, 16)])
                return carry2

            lax.fori_loop(0, CHUNK // 128, add8, 0)
            pltpu.sync_copy(xbuf, o_hbm.at[b, pl.ds(off, CHUNK)])
        return carry

    lax.fori_loop(0, NCH, chunk_loop, 0)


def kernel(x, table):
    xf = x.reshape(B, T * D)
    tf = table.reshape(T * D)
    k = pl.kernel(
        _sc_body,
        mesh=plsc.VectorSubcoreMesh(core_axis_name="c", subcore_axis_name="s"),
        out_type=jax.ShapeDtypeStruct((B, T * D), jnp.float32),
        scratch_types=[
            pltpu.VMEM((CHUNK,), jnp.float32),
            pltpu.VMEM((CHUNK,), jnp.float32),
        ],
    )
    return k(xf, tf).reshape(B, T, D)
